# Initial kernel scaffold; baseline (speedup 1.0000x reference)
import jax
import jax.numpy as jnp
from jax.experimental import pallas as pl


def kernel(x, gate_W, gate_b, W1, b1, W2, b2, gamma):
    raise NotImplementedError("write your pallas kernel here")


# speedup vs baseline: 2.0265x; 2.0265x over previous
import functools

import jax
import jax.numpy as jnp
from jax.experimental import pallas as pl
from jax.experimental.pallas import tpu as pltpu

LANES = 128


def _gelu(v):
    return 0.5 * v * (1.0 + jax.lax.erf(v * (2.0 ** -0.5)))


def _gate_body(x_ref, gw_ref, gb_ref, tv_ref, idx_ref, loss_ref, imp_ref, *, E):
    t = pl.program_id(0)
    logits = jnp.dot(x_ref[...], gw_ref[...], preferred_element_type=jnp.float32)
    logits = logits + gb_ref[...]
    lane = jax.lax.broadcasted_iota(jnp.int32, logits.shape, 1)
    valid = lane < E
    lm = jnp.where(valid, logits, -1e30)
    m = jnp.max(lm, axis=1, keepdims=True)
    p = jnp.where(valid, jnp.exp(lm - m), 0.0)
    p = p / jnp.sum(p, axis=1, keepdims=True)
    tv = jnp.max(p, axis=1, keepdims=True)
    idx = jnp.min(jnp.where(p == tv, lane, LANES), axis=1, keepdims=True)
    tv_ref[...] = tv
    idx_ref[...] = idx

    @pl.when(t == 0)
    def _():
        imp_ref[...] = jnp.zeros_like(imp_ref)

    imp_ref[...] += jnp.sum(p, axis=0, keepdims=True)

    @pl.when(t == pl.num_programs(0) - 1)
    def _():
        imp = imp_ref[...]
        vmask = jax.lax.broadcasted_iota(jnp.int32, imp.shape, 1) < E
        mean = jnp.sum(imp) / E
        var = jnp.sum(jnp.where(vmask, (imp - mean) ** 2, 0.0)) / (E - 1)
        loss_ref[...] = var / (mean * mean + 1e-10) * jnp.ones_like(loss_ref)


def _expert_body(x_ref, w1_ref, b1_ref, w2_ref, b2_ref, tv_ref, idx_ref,
                 g_ref, out_ref, *, E, BT, D):
    e = pl.program_id(0)
    t = pl.program_id(1)
    xb = x_ref[...]
    h = jnp.dot(xb, w1_ref[0], preferred_element_type=jnp.float32) + b1_ref[0]
    h = _gelu(h)
    o = jnp.dot(h, w2_ref[0], preferred_element_type=jnp.float32) + b2_ref[0]
    contrib = jnp.where(idx_ref[...] == e, o, 0.0)
    sl = pl.ds(t * BT, BT)

    @pl.when(e == 0)
    def _():
        out_ref[sl, :] = contrib

    @pl.when(e != 0)
    def _():
        out_ref[sl, :] += contrib

    @pl.when(e == E - 1)
    def _():
        y = xb + out_ref[sl, :] * tv_ref[...]
        nrm = jnp.sqrt(jnp.sum(y * y, axis=1, keepdims=True))
        y_n = y / jnp.maximum(nrm, 1e-12) * g_ref[...] * (D ** 0.5)
        out_ref[sl, :] = _gelu(y_n)


def kernel(x, gate_W, gate_b, W1, b1, W2, b2, gamma):
    B, N, D = x.shape
    E, _, H = W1.shape
    x_flat = x.reshape(N, D)
    BT = 256
    T = N // BT

    gwp = jnp.zeros((D, LANES), jnp.float32).at[:, :E].set(gate_W)
    gbp = jnp.zeros((1, LANES), jnp.float32).at[0, :E].set(gate_b)

    tv, idx, loss = pl.pallas_call(
        functools.partial(_gate_body, E=E),
        grid=(T,),
        in_specs=[
            pl.BlockSpec((BT, D), lambda t: (t, 0)),
            pl.BlockSpec((D, LANES), lambda t: (0, 0)),
            pl.BlockSpec((1, LANES), lambda t: (0, 0)),
        ],
        out_specs=[
            pl.BlockSpec((BT, 1), lambda t: (t, 0)),
            pl.BlockSpec((BT, 1), lambda t: (t, 0)),
            pl.BlockSpec((1, 1), lambda t: (0, 0)),
        ],
        out_shape=[
            jax.ShapeDtypeStruct((N, 1), jnp.float32),
            jax.ShapeDtypeStruct((N, 1), jnp.int32),
            jax.ShapeDtypeStruct((1, 1), jnp.float32),
        ],
        scratch_shapes=[pltpu.VMEM((1, LANES), jnp.float32)],
    )(x_flat, gwp, gbp)

    out = pl.pallas_call(
        functools.partial(_expert_body, E=E, BT=BT, D=D),
        grid=(E, T),
        in_specs=[
            pl.BlockSpec((BT, D), lambda e, t: (t, 0)),
            pl.BlockSpec((1, D, H), lambda e, t: (e, 0, 0)),
            pl.BlockSpec((1, 1, H), lambda e, t: (e, 0, 0)),
            pl.BlockSpec((1, H, D), lambda e, t: (e, 0, 0)),
            pl.BlockSpec((1, 1, D), lambda e, t: (e, 0, 0)),
            pl.BlockSpec((BT, 1), lambda e, t: (t, 0)),
            pl.BlockSpec((BT, 1), lambda e, t: (t, 0)),
            pl.BlockSpec((1, D), lambda e, t: (0, 0)),
        ],
        out_specs=pl.BlockSpec((N, D), lambda e, t: (0, 0)),
        out_shape=jax.ShapeDtypeStruct((N, D), jnp.float32),
    )(x_flat, W1, b1.reshape(E, 1, H), W2, b2.reshape(E, 1, D), tv, idx,
      gamma.reshape(1, D))

    return out.reshape(B, N, D), loss.reshape(())
